# baseline (speedup 1.0000x reference)
import functools

import jax
import jax.numpy as jnp
from jax import lax
from jax.experimental import pallas as pl
from jax.experimental.pallas import tpu as pltpu
from jax.experimental.pallas import tpu_sc as plsc

V = 8
N = 16384
D = 64
K = 1024
BN = 2048
NB = N // BN

NC = 2
NS = 16
NW = NC * NS
BW = (V * N) // NW
CH = 512
NCH = BW // CH


def _make_idx_body(v_base):
    def _idx_body(xt_ref, wt_ref, out_ref):
        v = pl.program_id(0) + v_base
        xt = xt_ref[0]
        wt = wt_ref[0]
        scores2 = jnp.dot(-2.0 * wt, xt, preferred_element_type=jnp.float32)
        xsq = jnp.sum(xt * xt, axis=0, keepdims=True)
        wsq = jnp.sum(wt * wt, axis=1, keepdims=True)
        dist = (xsq + scores2) + wsq
        m = jnp.min(dist, axis=0, keepdims=True)
        iota = jax.lax.broadcasted_iota(jnp.int32, (K, BN), 0).astype(jnp.float32)
        idxf = jnp.min(jnp.where(dist == m, iota, float(K)), axis=0)
        out_ref[0, 0] = idxf.astype(jnp.int32) + v * K

    return _idx_body


def _vq_idx(inputs_t, emb_t, v_base):
    nv = inputs_t.shape[0]
    return pl.pallas_call(
        _make_idx_body(v_base),
        grid=(nv, NB),
        in_specs=[
            pl.BlockSpec((1, D, BN), lambda v, n: (v, 0, n)),
            pl.BlockSpec((1, K, D), lambda v, n: (v, 0, 0)),
        ],
        out_specs=pl.BlockSpec((1, 1, BN), lambda v, n: (v * NB + n, 0, 0)),
        out_shape=jax.ShapeDtypeStruct((nv * NB, 1, BN), jnp.int32),
    )(inputs_t, emb_t)


_SC_MESH = plsc.VectorSubcoreMesh(core_axis_name="c", subcore_axis_name="s")


def _make_sc_gather(nrows):
    bw = nrows // NW
    nch = bw // CH

    @functools.partial(
        pl.kernel,
        out_type=jax.ShapeDtypeStruct((nrows, 2 * D), jnp.float32),
        mesh=_SC_MESH,
        scratch_types=[
            pltpu.VMEM((CH,), jnp.int32),
            pltpu.VMEM((CH, 2 * D), jnp.float32),
            pltpu.SemaphoreType.DMA,
        ],
    )
    def _sc_gather(table_hbm, idx_hbm, out_hbm, idx_v, rows_v, sem):
        wid = lax.axis_index("s") * NC + lax.axis_index("c")
        base = wid * bw

        def body(i, carry):
            off = base + i * CH
            pltpu.sync_copy(idx_hbm.at[pl.ds(off, CH)], idx_v)
            pltpu.async_copy(table_hbm.at[idx_v], rows_v, sem).wait()
            pltpu.sync_copy(rows_v, out_hbm.at[pl.ds(off, CH)])
            return carry

        lax.fori_loop(0, nch, body, 0)

    return _sc_gather


VH = V // 2
_sc_gather_half = _make_sc_gather(VH * N)


def kernel(inputs, embeddings):
    emb_t = jnp.transpose(embeddings, (0, 2, 1))
    inputs_t = jnp.transpose(inputs, (0, 2, 1))
    table = jnp.pad(emb_t.reshape(V * K, D), ((0, 0), (0, D)))
    qs = []
    for h in range(2):
        sl = slice(h * VH, (h + 1) * VH)
        idx3 = _vq_idx(inputs_t[sl], emb_t[sl], h * VH)
        qs.append(_sc_gather_half(table, idx3.reshape(VH * N)))
    q = jnp.concatenate(qs, axis=0)
    return q[:, :D].reshape(V, N, D)


# speedup vs baseline: 1.1523x; 1.1523x over previous
import functools

import jax
import jax.numpy as jnp
from jax import lax
from jax.experimental import pallas as pl
from jax.experimental.pallas import tpu as pltpu
from jax.experimental.pallas import tpu_sc as plsc

V = 8
N = 16384
D = 64
K = 1024
BN = 2048
NB = N // BN

NC = 2
NS = 16
NW = NC * NS
BW = (V * N) // NW
CH = 128
NCH = BW // CH


def _make_idx_body(v_base):
    def _idx_body(xt_ref, wt_ref, out_ref):
        v = pl.program_id(0) + v_base
        xt = xt_ref[0]
        wt = wt_ref[0]
        scores2 = jnp.dot(-2.0 * wt, xt, preferred_element_type=jnp.float32)
        xsq = jnp.sum(xt * xt, axis=0, keepdims=True)
        wsq = jnp.sum(wt * wt, axis=1, keepdims=True)
        dist = (xsq + scores2) + wsq
        m = jnp.min(dist, axis=0, keepdims=True)
        iota = jax.lax.broadcasted_iota(jnp.int32, (K, BN), 0).astype(jnp.float32)
        idxf = jnp.min(jnp.where(dist == m, iota, float(K)), axis=0)
        out_ref[0, 0] = idxf.astype(jnp.int32) + v * K

    return _idx_body


def _vq_idx(inputs_t, emb_t, v_base):
    nv = inputs_t.shape[0]
    return pl.pallas_call(
        _make_idx_body(v_base),
        grid=(nv, NB),
        in_specs=[
            pl.BlockSpec((1, D, BN), lambda v, n: (v, 0, n)),
            pl.BlockSpec((1, K, D), lambda v, n: (v, 0, 0)),
        ],
        out_specs=pl.BlockSpec((1, 1, BN), lambda v, n: (v * NB + n, 0, 0)),
        out_shape=jax.ShapeDtypeStruct((nv * NB, 1, BN), jnp.int32),
    )(inputs_t, emb_t)


_SC_MESH = plsc.VectorSubcoreMesh(core_axis_name="c", subcore_axis_name="s")


@functools.partial(
    pl.kernel,
    out_type=jax.ShapeDtypeStruct((V * N, D), jnp.float32),
    mesh=_SC_MESH,
    scratch_types=[
        pltpu.VMEM((BW,), jnp.int32),
        pltpu.VMEM((CH, 2 * D), jnp.float32),
        pltpu.VMEM((CH, 2 * D), jnp.float32),
        pltpu.VMEM((CH, D), jnp.float32),
        pltpu.VMEM((CH, D), jnp.float32),
        pltpu.SemaphoreType.DMA,
        pltpu.SemaphoreType.DMA,
        pltpu.SemaphoreType.DMA,
        pltpu.SemaphoreType.DMA,
    ],
)
def _sc_gather(table_hbm, idx_hbm, out_hbm, idx_v, rows0, rows1,
               comp0, comp1, semg0, semg1, semw0, semw1):
    wid = lax.axis_index("s") * NC + lax.axis_index("c")
    base = wid * BW
    rows = (rows0, rows1)
    comp = (comp0, comp1)
    semg = (semg0, semg1)
    semw = (semw0, semw1)

    pltpu.sync_copy(idx_hbm.at[pl.ds(base, BW)], idx_v)

    gh = [None] * NCH
    wh = [None] * NCH
    gh[0] = pltpu.async_copy(
        table_hbm.at[idx_v.at[pl.ds(0, CH)]], rows[0], semg[0])
    for i in range(NCH):
        cur = i % 2
        if i + 1 < NCH:
            gh[i + 1] = pltpu.async_copy(
                table_hbm.at[idx_v.at[pl.ds((i + 1) * CH, CH)]],
                rows[(i + 1) % 2], semg[(i + 1) % 2])
        gh[i].wait()
        if i >= 2:
            wh[i - 2].wait()
        rcur = rows[cur]
        ccur = comp[cur]

        def crow(t, c, rcur=rcur, ccur=ccur):
            for j in range(D // 16):
                ccur[t, pl.ds(j * 16, 16)] = rcur[t, pl.ds(j * 16, 16)]
            return c

        lax.fori_loop(0, CH, crow, 0)
        wh[i] = pltpu.async_copy(
            ccur, out_hbm.at[pl.ds(base + i * CH, CH)], semw[cur])
    wh[NCH - 2].wait()
    wh[NCH - 1].wait()


def kernel(inputs, embeddings):
    emb_t = jnp.transpose(embeddings, (0, 2, 1))
    inputs_t = jnp.transpose(inputs, (0, 2, 1))
    table = jnp.pad(emb_t.reshape(V * K, D), ((0, 0), (0, D)))
    idx3 = _vq_idx(inputs_t, emb_t, 0)
    q = _sc_gather(table, idx3.reshape(V * N))
    return q.reshape(V, N, D)


# speedup vs baseline: 1.1768x; 1.0212x over previous
import functools

import jax
import jax.numpy as jnp
from jax import lax
from jax.experimental import pallas as pl
from jax.experimental.pallas import tpu as pltpu
from jax.experimental.pallas import tpu_sc as plsc

V = 8
N = 16384
D = 64
K = 1024
BN = 4096
NB = N // BN

NC = 2
NS = 16
NW = NC * NS
BW = (V * N) // NW
CH = 128
NCH = BW // CH


def _make_idx_body(v_base):
    def _idx_body(xt_ref, wt_ref, out_ref):
        v = pl.program_id(0) + v_base
        xt = xt_ref[0]
        wt = wt_ref[0]
        scores2 = jnp.dot(-2.0 * wt, xt, preferred_element_type=jnp.float32)
        xsq = jnp.sum(xt * xt, axis=0, keepdims=True)
        wsq = jnp.sum(wt * wt, axis=1, keepdims=True)
        dist = (xsq + scores2) + wsq
        m = jnp.min(dist, axis=0, keepdims=True)
        iota = jax.lax.broadcasted_iota(jnp.int32, (K, BN), 0).astype(jnp.float32)
        idxf = jnp.min(jnp.where(dist == m, iota, float(K)), axis=0)
        out_ref[0, 0] = idxf.astype(jnp.int32) + v * K

    return _idx_body


def _vq_idx(inputs_t, emb_t, v_base):
    nv = inputs_t.shape[0]
    return pl.pallas_call(
        _make_idx_body(v_base),
        grid=(nv, NB),
        in_specs=[
            pl.BlockSpec((1, D, BN), lambda v, n: (v, 0, n)),
            pl.BlockSpec((1, K, D), lambda v, n: (v, 0, 0)),
        ],
        out_specs=pl.BlockSpec((1, 1, BN), lambda v, n: (v * NB + n, 0, 0)),
        out_shape=jax.ShapeDtypeStruct((nv * NB, 1, BN), jnp.int32),
    )(inputs_t, emb_t)


_SC_MESH = plsc.VectorSubcoreMesh(core_axis_name="c", subcore_axis_name="s")


@functools.partial(
    pl.kernel,
    out_type=jax.ShapeDtypeStruct((V * N, D), jnp.float32),
    mesh=_SC_MESH,
    scratch_types=[
        pltpu.VMEM((BW,), jnp.int32),
        pltpu.VMEM((CH, 2 * D), jnp.float32),
        pltpu.VMEM((CH, 2 * D), jnp.float32),
        pltpu.VMEM((CH, D), jnp.float32),
        pltpu.VMEM((CH, D), jnp.float32),
        pltpu.SemaphoreType.DMA,
        pltpu.SemaphoreType.DMA,
        pltpu.SemaphoreType.DMA,
        pltpu.SemaphoreType.DMA,
    ],
)
def _sc_gather(table_hbm, idx_hbm, out_hbm, idx_v, rows0, rows1,
               comp0, comp1, semg0, semg1, semw0, semw1):
    wid = lax.axis_index("s") * NC + lax.axis_index("c")
    base = wid * BW
    rows = (rows0, rows1)
    comp = (comp0, comp1)
    semg = (semg0, semg1)
    semw = (semw0, semw1)

    pltpu.sync_copy(idx_hbm.at[pl.ds(base, BW)], idx_v)

    gh = [None] * NCH
    wh = [None] * NCH
    gh[0] = pltpu.async_copy(
        table_hbm.at[idx_v.at[pl.ds(0, CH)]], rows[0], semg[0])
    for i in range(NCH):
        cur = i % 2
        if i + 1 < NCH:
            gh[i + 1] = pltpu.async_copy(
                table_hbm.at[idx_v.at[pl.ds((i + 1) * CH, CH)]],
                rows[(i + 1) % 2], semg[(i + 1) % 2])
        gh[i].wait()
        if i >= 2:
            wh[i - 2].wait()
        rcur = rows[cur]
        ccur = comp[cur]

        def crow(t, c, rcur=rcur, ccur=ccur):
            for j in range(D // 16):
                ccur[t, pl.ds(j * 16, 16)] = rcur[t, pl.ds(j * 16, 16)]
            return c

        lax.fori_loop(0, CH, crow, 0)
        wh[i] = pltpu.async_copy(
            ccur, out_hbm.at[pl.ds(base + i * CH, CH)], semw[cur])
    wh[NCH - 2].wait()
    wh[NCH - 1].wait()


def kernel(inputs, embeddings):
    emb_t = jnp.transpose(embeddings, (0, 2, 1))
    inputs_t = jnp.transpose(inputs, (0, 2, 1))
    table = jnp.pad(emb_t.reshape(V * K, D), ((0, 0), (0, D)))
    idx3 = _vq_idx(inputs_t, emb_t, 0)
    q = _sc_gather(table, idx3.reshape(V * N))
    return q.reshape(V, N, D)


# speedup vs baseline: 1.1918x; 1.0128x over previous
import functools

import jax
import jax.numpy as jnp
from jax import lax
from jax.experimental import pallas as pl
from jax.experimental.pallas import tpu as pltpu
from jax.experimental.pallas import tpu_sc as plsc

V = 8
N = 16384
D = 64
K = 1024
BN = 8192
NB = N // BN

NC = 2
NS = 16
NW = NC * NS
BW = (V * N) // NW
CH = 128
NCH = BW // CH


def _make_idx_body(v_base):
    def _idx_body(xt_ref, wt_ref, out_ref):
        v = pl.program_id(0) + v_base
        xt = xt_ref[0]
        wt = wt_ref[0]
        scores2 = jnp.dot(-2.0 * wt, xt, preferred_element_type=jnp.float32)
        xsq = jnp.sum(xt * xt, axis=0, keepdims=True)
        wsq = jnp.sum(wt * wt, axis=1, keepdims=True)
        dist = (xsq + scores2) + wsq
        m = jnp.min(dist, axis=0, keepdims=True)
        iota = jax.lax.broadcasted_iota(jnp.int32, (K, BN), 0).astype(jnp.float32)
        idxf = jnp.min(jnp.where(dist == m, iota, float(K)), axis=0)
        out_ref[0, 0] = idxf.astype(jnp.int32) + v * K

    return _idx_body


def _vq_idx(inputs_t, emb_t, v_base):
    nv = inputs_t.shape[0]
    return pl.pallas_call(
        _make_idx_body(v_base),
        grid=(nv, NB),
        in_specs=[
            pl.BlockSpec((1, D, BN), lambda v, n: (v, 0, n)),
            pl.BlockSpec((1, K, D), lambda v, n: (v, 0, 0)),
        ],
        out_specs=pl.BlockSpec((1, 1, BN), lambda v, n: (v * NB + n, 0, 0)),
        out_shape=jax.ShapeDtypeStruct((nv * NB, 1, BN), jnp.int32),
    )(inputs_t, emb_t)


_SC_MESH = plsc.VectorSubcoreMesh(core_axis_name="c", subcore_axis_name="s")


@functools.partial(
    pl.kernel,
    out_type=jax.ShapeDtypeStruct((V * N, D), jnp.float32),
    mesh=_SC_MESH,
    scratch_types=[
        pltpu.VMEM((BW,), jnp.int32),
        pltpu.VMEM((CH, 2 * D), jnp.float32),
        pltpu.VMEM((CH, 2 * D), jnp.float32),
        pltpu.VMEM((CH, D), jnp.float32),
        pltpu.VMEM((CH, D), jnp.float32),
        pltpu.SemaphoreType.DMA,
        pltpu.SemaphoreType.DMA,
        pltpu.SemaphoreType.DMA,
        pltpu.SemaphoreType.DMA,
    ],
)
def _sc_gather(table_hbm, idx_hbm, out_hbm, idx_v, rows0, rows1,
               comp0, comp1, semg0, semg1, semw0, semw1):
    wid = lax.axis_index("s") * NC + lax.axis_index("c")
    base = wid * BW
    rows = (rows0, rows1)
    comp = (comp0, comp1)
    semg = (semg0, semg1)
    semw = (semw0, semw1)

    pltpu.sync_copy(idx_hbm.at[pl.ds(base, BW)], idx_v)

    gh = [None] * NCH
    wh = [None] * NCH
    gh[0] = pltpu.async_copy(
        table_hbm.at[idx_v.at[pl.ds(0, CH)]], rows[0], semg[0])
    for i in range(NCH):
        cur = i % 2
        if i + 1 < NCH:
            gh[i + 1] = pltpu.async_copy(
                table_hbm.at[idx_v.at[pl.ds((i + 1) * CH, CH)]],
                rows[(i + 1) % 2], semg[(i + 1) % 2])
        gh[i].wait()
        if i >= 2:
            wh[i - 2].wait()
        rcur = rows[cur]
        ccur = comp[cur]

        def crow(t, c, rcur=rcur, ccur=ccur):
            for j in range(D // 16):
                ccur[t, pl.ds(j * 16, 16)] = rcur[t, pl.ds(j * 16, 16)]
            return c

        lax.fori_loop(0, CH, crow, 0)
        wh[i] = pltpu.async_copy(
            ccur, out_hbm.at[pl.ds(base + i * CH, CH)], semw[cur])
    wh[NCH - 2].wait()
    wh[NCH - 1].wait()


def kernel(inputs, embeddings):
    emb_t = jnp.transpose(embeddings, (0, 2, 1))
    inputs_t = jnp.transpose(inputs, (0, 2, 1))
    table = jnp.pad(emb_t.reshape(V * K, D), ((0, 0), (0, D)))
    idx3 = _vq_idx(inputs_t, emb_t, 0)
    q = _sc_gather(table, idx3.reshape(V * N))
    return q.reshape(V, N, D)
